# baseline (speedup 1.0000x reference)
import functools


import jax
import jax.numpy as jnp
from jax import lax
from jax.experimental import pallas as pl
from jax.experimental.pallas import tpu as pltpu
from jax.experimental.pallas import tpu_sc as plsc

NBUF = 4
LANES = 16
CHB = 128


@functools.lru_cache(maxsize=None)
def _make_sc_kernel(B, S, V, D):
    info = plsc.get_sparse_core_info()
    NC, NS = info.num_cores, info.num_subcores
    NW = NC * NS
    BT = B // CHB
    DT = D // 8
    KD = D // LANES
    NG = -(-S // NBUF)
    TW = DT * CHB

    assert BT == NW and D % LANES == 0 and CHB % LANES == 0

    mesh = plsc.VectorSubcoreMesh(core_axis_name="c", subcore_axis_name="s")

    @functools.partial(
        pl.kernel,
        mesh=mesh,
        compiler_params=pltpu.CompilerParams(
            use_tc_tiling_on_sc=False, needs_layout_passes=False),
        out_type=jax.ShapeDtypeStruct((S, DT * BT * TW), jnp.float32),
        scratch_types=[
            pltpu.VMEM((S, CHB), jnp.int32),
            pltpu.VMEM((S, D), jnp.float32),
        ]
        + [pltpu.VMEM((CHB, D), jnp.float32) for _ in range(NBUF)]
        + [pltpu.VMEM((DT * TW,), jnp.float32) for _ in range(NBUF)]
        + [pltpu.SemaphoreType.DMA for _ in range(2 * NBUF)],
    )
    def k(tok_hbm, idx_hbm, pos_hbm, out_hbm, idx_v, pos_v, *bufs_and_sems):
        gbufs = bufs_and_sems[0:NBUF]
        tbufs = bufs_and_sems[NBUF:2 * NBUF]
        gsems = bufs_and_sems[2 * NBUF:3 * NBUF]
        ssems = bufs_and_sems[3 * NBUF:4 * NBUF]

        wid = lax.axis_index("s") * NC + lax.axis_index("c")

        pltpu.sync_copy(idx_hbm.at[:, wid], idx_v)
        pltpu.sync_copy(pos_hbm, pos_v)

        iota = lax.iota(jnp.int32, LANES)
        cols = [[d0s * LANES + ((iota + kk) & (LANES - 1))
                 for kk in range(LANES)] for d0s in range(KD)]
        dcons = [[(c >> 3) * TW + (c & 7) * CHB + iota for c in row]
                 for row in cols]

        def gather(s, g):
            return pltpu.make_async_copy(
                tok_hbm.at[idx_v.at[s]], gbufs[g], gsems[g])

        def scatters(s, t):
            return [
                pltpu.make_async_copy(
                    tbufs[t].at[pl.ds(dt * TW, TW)],
                    out_hbm.at[s, pl.ds(dt * NW * TW + wid * TW, TW)],
                    ssems[t])
                for dt in range(DT)
            ]

        for g in range(NBUF):
            gather(g, g).start()

        def outer(i, carry):
            s0 = i * NBUF
            for b in range(NBUF):
                s = s0 + b

                @pl.when(s < S)
                def _(s=s, b=b):
                    gather(s, b).wait()

                    @pl.when(s >= NBUF)
                    def _():
                        for c in scatters(s - NBUF, b):
                            c.wait()

                    full_s = jnp.full((LANES,), s, dtype=jnp.int32)
                    for d0s in range(KD):
                        posp = [
                            plsc.load_gather(pos_v, [full_s, cols[d0s][kk]])
                            for kk in range(LANES)
                        ]

                        @plsc.parallel_loop(0, CHB, step=LANES, unroll=4)
                        def blk(r0, _b=b, _d0s=d0s, _posp=posp):
                            rf = jnp.full((LANES,), r0, dtype=jnp.int32)
                            rv = rf + iota
                            for kk in range(LANES):
                                v = plsc.load_gather(
                                    gbufs[_b], [rv, cols[_d0s][kk]])
                                plsc.store_scatter(
                                    tbufs[_b], [dcons[_d0s][kk] + rf],
                                    v + _posp[kk])

                    for c in scatters(s, b):
                        c.start()

                    @pl.when(s + NBUF < S)
                    def _():
                        gather(s + NBUF, b).start()
            return carry

        lax.fori_loop(0, NG, outer, 0)

        for b in range(NBUF):
            last = (S - 1 - b) // NBUF * NBUF + b
            for c in scatters(last, b):
                c.wait()

    return k, NW


@functools.lru_cache(maxsize=None)
def _make_transpose_kernel(V, D):
    info = plsc.get_sparse_core_info()
    NC, NS = info.num_cores, info.num_subcores
    NW = NC * NS
    DT = D // 8
    KD = D // LANES
    TPB = 8 * CHB * DT
    NBLK = V // CHB
    TAIL = V - NBLK * CHB
    NJ = -(-NBLK // NW)

    mesh = plsc.VectorSubcoreMesh(core_axis_name="c", subcore_axis_name="s")

    @functools.partial(
        pl.kernel,
        mesh=mesh,
        compiler_params=pltpu.CompilerParams(
            use_tc_tiling_on_sc=True, needs_layout_passes=False),
        out_type=jax.ShapeDtypeStruct((V * D,), jnp.float32),
        scratch_types=[pltpu.VMEM((TAIL * D,), jnp.float32)]
        + [pltpu.VMEM((DT, 8, CHB), jnp.float32) for _ in range(NBUF)]
        + [pltpu.VMEM((TPB,), jnp.float32) for _ in range(NBUF)]
        + [pltpu.SemaphoreType.DMA for _ in range(2 * NBUF)],
    )
    def k1(tt_hbm, tail_hbm, out_hbm, tailv, *bufs_and_sems):
        vbufs = bufs_and_sems[0:NBUF]
        tbufs = bufs_and_sems[NBUF:2 * NBUF]
        gsems = bufs_and_sems[2 * NBUF:3 * NBUF]
        ssems = bufs_and_sems[3 * NBUF:4 * NBUF]

        wid = lax.axis_index("s") * NC + lax.axis_index("c")

        @pl.when(wid == 0)
        def _():
            pltpu.sync_copy(tail_hbm, tailv)
            pltpu.sync_copy(tailv, out_hbm.at[pl.ds(NBLK * CHB * D, TAIL * D)])

        iota = lax.iota(jnp.int32, LANES)
        cols = [[d0s * LANES + ((iota + kk) & (LANES - 1))
                 for kk in range(LANES)] for d0s in range(KD)]
        dtvs = [[c >> 3 for c in row] for row in cols]
        rvs = [[c & 7 for c in row] for row in cols]
        scons = [[iota * D + c for c in row] for row in cols]

        def gathers(blk, g):
            return [
                pltpu.make_async_copy(
                    tt_hbm.at[pl.ds(8 * dt, 8), pl.ds(blk * CHB, CHB)],
                    vbufs[g].at[dt], gsems[g])
                for dt in range(DT)
            ]

        def scatter(blk, t):
            return pltpu.make_async_copy(
                tbufs[t], out_hbm.at[pl.ds(blk * CHB * D, TPB)], ssems[t])

        def blk_of(j, b):
            return (j * NBUF + b) * NW + wid

        for g in range(NBUF):
            @pl.when(blk_of(0, g) < NBLK)
            def _(g=g):
                for c in gathers(blk_of(0, g), g):
                    c.start()

        def outer(j, carry):
            for b in range(NBUF):
                blk = blk_of(j, b)

                @pl.when(blk < NBLK)
                def _(blk=blk, b=b, j=j):
                    for c in gathers(blk, b):
                        c.wait()

                    @pl.when(j > 0)
                    def _():
                        scatter(blk - NBUF * NW, b).wait()

                    for d0s in range(KD):
                        @plsc.parallel_loop(0, CHB, step=LANES, unroll=4)
                        def vblk(v0, _b=b, _d0s=d0s):
                            vf = jnp.full((LANES,), v0, dtype=jnp.int32)
                            cv = vf + iota
                            for kk in range(LANES):
                                x = plsc.load_gather(
                                    vbufs[_b],
                                    [dtvs[_d0s][kk], rvs[_d0s][kk], cv])
                                plsc.store_scatter(
                                    tbufs[_b],
                                    [scons[_d0s][kk] + vf * D], x)

                    scatter(blk, b).start()
                    nblk = blk + NBUF * NW

                    @pl.when(nblk < NBLK)
                    def _():
                        for c in gathers(nblk, b):
                            c.start()
            return carry

        lax.fori_loop(0, NJ, outer, 0)

        for b in range(NBUF):
            jmax = (NBLK - 1 - wid - b * NW) // (NBUF * NW)
            scatter((jmax * NBUF + b) * NW + wid, b).wait()

    return k1


def kernel(seq, tok_table, pos_table):
    B, S = seq.shape
    V, D = tok_table.shape
    k, NW = _make_sc_kernel(B, S, V, D)
    k1 = _make_transpose_kernel(V, D)
    NBLK = V // CHB
    tail = tok_table[NBLK * CHB:].reshape(-1)
    tok_flat = k1(tok_table.T, tail)
    idx = seq.astype(jnp.int32).T.reshape(S, NW, CHB)
    out4 = k(tok_flat.reshape(V, D), idx, pos_table)
    out = (
        out4.reshape(S, D // 8, B // CHB, 8, CHB)
        .transpose(2, 4, 0, 1, 3)
        .reshape(B, S, D)
    )
    return out


# speedup vs baseline: 1.0236x; 1.0236x over previous
import functools


import jax
import jax.numpy as jnp
from jax import lax
from jax.experimental import pallas as pl
from jax.experimental.pallas import tpu as pltpu
from jax.experimental.pallas import tpu_sc as plsc

NBUF = 3
LANES = 16
CHB = 128


@functools.lru_cache(maxsize=None)
def _make_sc_kernel(B, S, V, D):
    info = plsc.get_sparse_core_info()
    NC, NS = info.num_cores, info.num_subcores
    NW = NC * NS
    BT = B // CHB
    DT = D // 8
    KD = D // LANES
    NG = -(-S // NBUF)
    TW = DT * CHB

    assert BT == NW and D % LANES == 0 and CHB % LANES == 0

    mesh = plsc.VectorSubcoreMesh(core_axis_name="c", subcore_axis_name="s")

    @functools.partial(
        pl.kernel,
        mesh=mesh,
        compiler_params=pltpu.CompilerParams(
            use_tc_tiling_on_sc=False, needs_layout_passes=False),
        out_type=jax.ShapeDtypeStruct((S, DT * BT * TW), jnp.float32),
        scratch_types=[
            pltpu.VMEM((S, CHB), jnp.int32),
            pltpu.VMEM((S, D), jnp.float32),
        ]
        + [pltpu.VMEM((CHB, D), jnp.float32) for _ in range(NBUF)]
        + [pltpu.VMEM((DT * TW,), jnp.float32) for _ in range(NBUF)]
        + [pltpu.SemaphoreType.DMA for _ in range(2 * NBUF)],
    )
    def k(tok_hbm, idx_hbm, pos_hbm, out_hbm, idx_v, pos_v, *bufs_and_sems):
        gbufs = bufs_and_sems[0:NBUF]
        tbufs = bufs_and_sems[NBUF:2 * NBUF]
        gsems = bufs_and_sems[2 * NBUF:3 * NBUF]
        ssems = bufs_and_sems[3 * NBUF:4 * NBUF]

        wid = lax.axis_index("s") * NC + lax.axis_index("c")

        pltpu.sync_copy(idx_hbm.at[:, wid], idx_v)
        pltpu.sync_copy(pos_hbm, pos_v)

        iota = lax.iota(jnp.int32, LANES)
        cols = [[d0s * LANES + ((iota + kk) & (LANES - 1))
                 for kk in range(LANES)] for d0s in range(KD)]
        dcons = [[(c >> 3) * TW + (c & 7) * CHB + iota for c in row]
                 for row in cols]

        def gather(s, g):
            return pltpu.make_async_copy(
                tok_hbm.at[idx_v.at[s]], gbufs[g], gsems[g])

        def scatters(s, t):
            return [
                pltpu.make_async_copy(
                    tbufs[t].at[pl.ds(dt * TW, TW)],
                    out_hbm.at[s, pl.ds(dt * NW * TW + wid * TW, TW)],
                    ssems[t])
                for dt in range(DT)
            ]

        for g in range(NBUF):
            gather(g, g).start()

        def outer(i, carry):
            s0 = i * NBUF
            for b in range(NBUF):
                s = s0 + b

                @pl.when(s < S)
                def _(s=s, b=b):
                    gather(s, b).wait()

                    @pl.when(s >= NBUF)
                    def _():
                        for c in scatters(s - NBUF, b):
                            c.wait()

                    full_s = jnp.full((LANES,), s, dtype=jnp.int32)
                    for d0s in range(KD):
                        posp = [
                            plsc.load_gather(pos_v, [full_s, cols[d0s][kk]])
                            for kk in range(LANES)
                        ]

                        @plsc.parallel_loop(0, CHB, step=LANES, unroll=4)
                        def blk(r0, _b=b, _d0s=d0s, _posp=posp):
                            rf = jnp.full((LANES,), r0, dtype=jnp.int32)
                            rv = rf + iota
                            for kk in range(LANES):
                                v = plsc.load_gather(
                                    gbufs[_b], [rv, cols[_d0s][kk]])
                                plsc.store_scatter(
                                    tbufs[_b], [dcons[_d0s][kk] + rf],
                                    v + _posp[kk])

                    for c in scatters(s, b):
                        c.start()

                    @pl.when(s + NBUF < S)
                    def _():
                        gather(s + NBUF, b).start()
            return carry

        lax.fori_loop(0, NG, outer, 0)

        for b in range(NBUF):
            last = (S - 1 - b) // NBUF * NBUF + b
            for c in scatters(last, b):
                c.wait()

    return k, NW


@functools.lru_cache(maxsize=None)
def _make_transpose_kernel(V, D):
    info = plsc.get_sparse_core_info()
    NC, NS = info.num_cores, info.num_subcores
    NW = NC * NS
    DT = D // 8
    KD = D // LANES
    TPB = 8 * CHB * DT
    NBLK = V // CHB
    TAIL = V - NBLK * CHB
    NJ = -(-NBLK // NW)

    mesh = plsc.VectorSubcoreMesh(core_axis_name="c", subcore_axis_name="s")

    @functools.partial(
        pl.kernel,
        mesh=mesh,
        compiler_params=pltpu.CompilerParams(
            use_tc_tiling_on_sc=True, needs_layout_passes=False),
        out_type=jax.ShapeDtypeStruct((V * D,), jnp.float32),
        scratch_types=[pltpu.VMEM((TAIL * D,), jnp.float32)]
        + [pltpu.VMEM((DT, 8, CHB), jnp.float32) for _ in range(NBUF)]
        + [pltpu.VMEM((TPB,), jnp.float32) for _ in range(NBUF)]
        + [pltpu.SemaphoreType.DMA for _ in range(2 * NBUF)],
    )
    def k1(tt_hbm, tail_hbm, out_hbm, tailv, *bufs_and_sems):
        vbufs = bufs_and_sems[0:NBUF]
        tbufs = bufs_and_sems[NBUF:2 * NBUF]
        gsems = bufs_and_sems[2 * NBUF:3 * NBUF]
        ssems = bufs_and_sems[3 * NBUF:4 * NBUF]

        wid = lax.axis_index("s") * NC + lax.axis_index("c")

        @pl.when(wid == 0)
        def _():
            pltpu.sync_copy(tail_hbm, tailv)
            pltpu.sync_copy(tailv, out_hbm.at[pl.ds(NBLK * CHB * D, TAIL * D)])

        iota = lax.iota(jnp.int32, LANES)
        cols = [[d0s * LANES + ((iota + kk) & (LANES - 1))
                 for kk in range(LANES)] for d0s in range(KD)]
        dtvs = [[c >> 3 for c in row] for row in cols]
        rvs = [[c & 7 for c in row] for row in cols]
        scons = [[iota * D + c for c in row] for row in cols]

        def gathers(blk, g):
            return [
                pltpu.make_async_copy(
                    tt_hbm.at[pl.ds(8 * dt, 8), pl.ds(blk * CHB, CHB)],
                    vbufs[g].at[dt], gsems[g])
                for dt in range(DT)
            ]

        def scatter(blk, t):
            return pltpu.make_async_copy(
                tbufs[t], out_hbm.at[pl.ds(blk * CHB * D, TPB)], ssems[t])

        def blk_of(j, b):
            return (j * NBUF + b) * NW + wid

        for g in range(NBUF):
            @pl.when(blk_of(0, g) < NBLK)
            def _(g=g):
                for c in gathers(blk_of(0, g), g):
                    c.start()

        def outer(j, carry):
            for b in range(NBUF):
                blk = blk_of(j, b)

                @pl.when(blk < NBLK)
                def _(blk=blk, b=b, j=j):
                    for c in gathers(blk, b):
                        c.wait()

                    @pl.when(j > 0)
                    def _():
                        scatter(blk - NBUF * NW, b).wait()

                    for d0s in range(KD):
                        @plsc.parallel_loop(0, CHB, step=LANES, unroll=4)
                        def vblk(v0, _b=b, _d0s=d0s):
                            vf = jnp.full((LANES,), v0, dtype=jnp.int32)
                            cv = vf + iota
                            for kk in range(LANES):
                                x = plsc.load_gather(
                                    vbufs[_b],
                                    [dtvs[_d0s][kk], rvs[_d0s][kk], cv])
                                plsc.store_scatter(
                                    tbufs[_b],
                                    [scons[_d0s][kk] + vf * D], x)

                    scatter(blk, b).start()
                    nblk = blk + NBUF * NW

                    @pl.when(nblk < NBLK)
                    def _():
                        for c in gathers(nblk, b):
                            c.start()
            return carry

        lax.fori_loop(0, NJ, outer, 0)

        for b in range(NBUF):
            jmax = (NBLK - 1 - wid - b * NW) // (NBUF * NW)
            scatter((jmax * NBUF + b) * NW + wid, b).wait()

    return k1


def kernel(seq, tok_table, pos_table):
    B, S = seq.shape
    V, D = tok_table.shape
    k, NW = _make_sc_kernel(B, S, V, D)
    k1 = _make_transpose_kernel(V, D)
    NBLK = V // CHB
    tail = tok_table[NBLK * CHB:].reshape(-1)
    tok_flat = k1(tok_table.T, tail)
    idx = seq.astype(jnp.int32).T.reshape(S, NW, CHB)
    out4 = k(tok_flat.reshape(V, D), idx, pos_table)
    out = (
        out4.reshape(S, D // 8, B // CHB, 8, CHB)
        .transpose(2, 4, 0, 1, 3)
        .reshape(B, S, D)
    )
    return out
